# baseline (speedup 1.0000x reference)
import functools

import jax
import jax.numpy as jnp
from jax import lax
from jax.experimental import pallas as pl
from jax.experimental.pallas import tpu as pltpu
from jax.experimental.pallas import tpu_sc as plsc

_VOCAB = 1000000
_D = 64
_B = 4096
_H = 50
_N = _B * _H
_SCALE = 8.0

_NC = 2
_NS = 16
_NW = _NC * _NS
_C = 128
_CHUNKS = _N // (_NW * _C)


@jax.jit
def _sc_embed(idx3, tabp):
    mesh = plsc.VectorSubcoreMesh(
        core_axis_name="c", subcore_axis_name="s",
        num_cores=_NC, num_subcores=_NS)

    @functools.partial(
        pl.kernel,
        out_type=jax.ShapeDtypeStruct((_N, _D), jnp.float32),
        mesh=mesh,
        scratch_types=[
            pltpu.VMEM((_CHUNKS, _C), jnp.int32),
            pltpu.VMEM((2, _C, 2 * _D), jnp.bfloat16),
            pltpu.VMEM((2, _C, _D), jnp.float32),
            pltpu.SemaphoreType.DMA,
            pltpu.SemaphoreType.DMA,
            pltpu.SemaphoreType.DMA,
            pltpu.SemaphoreType.DMA,
        ],
        compiler_params=pltpu.CompilerParams(
            use_tc_tiling_on_sc=False, needs_layout_passes=False),
    )
    def k(idx_hbm, tab_hbm, out_hbm, idx_v, rows_v, out_v,
          gsem0, gsem1, osem0, osem1):
        gsems = (gsem0, gsem1)
        osems = (osem0, osem1)
        wid = lax.axis_index("s") * _NC + lax.axis_index("c")
        pltpu.sync_copy(idx_hbm.at[wid], idx_v)

        rbase0 = pl.multiple_of(wid * (_CHUNKS * _C), _C)

        def start_gather(ck, b):
            pltpu.async_copy(tab_hbm.at[idx_v.at[ck]], rows_v.at[b], gsems[b])

        def wait_gather(ck, b):
            pltpu.make_async_copy(
                tab_hbm.at[idx_v.at[ck]], rows_v.at[b], gsems[b]).wait()

        def wb_slice(ck):
            rb = pl.multiple_of(rbase0 + ck * _C, _C)
            return out_hbm.at[pl.ds(rb, _C)]

        iota2 = lax.iota(jnp.int32, 16) * 2
        mhi = jnp.full((16,), -65536, jnp.int32)

        def extract(b):
            rows = rows_v.at[b]
            outb = out_v.at[b]

            @plsc.parallel_loop(0, _C, 1, unroll=4)
            def _(j):
                jv = jnp.full((16,), j, jnp.int32)
                for g in range(_D // 32):
                    v16 = rows[j, pl.ds(32 * g, 32)]
                    vi = plsc.bitcast(v16, jnp.int32)
                    lo = plsc.bitcast(vi << 16, jnp.float32) * _SCALE
                    hi = plsc.bitcast(vi & mhi, jnp.float32) * _SCALE
                    plsc.store_scatter(outb, [jv, iota2 + 32 * g], lo)
                    plsc.store_scatter(outb, [jv, iota2 + (32 * g + 1)], hi)

        start_gather(0, 0)
        start_gather(1, 1)

        @pl.loop(0, _CHUNKS // 2)
        def _(g2):
            for b in (0, 1):
                ck = 2 * g2 + b
                wait_gather(ck, b)

                @pl.when(ck >= 2)
                def _():
                    pltpu.make_async_copy(
                        out_v.at[b], wb_slice(ck - 2), osems[b]).wait()

                extract(b)
                pltpu.async_copy(out_v.at[b], wb_slice(ck), osems[b])

                @pl.when(ck + 2 < _CHUNKS)
                def _():
                    start_gather(ck + 2, b)

        pltpu.make_async_copy(
            out_v.at[0], wb_slice(_CHUNKS - 2), osems[0]).wait()
        pltpu.make_async_copy(
            out_v.at[1], wb_slice(_CHUNKS - 1), osems[1]).wait()

    return k(idx3, tabp)


def kernel(inputs, embeddings):
    idx3 = inputs.astype(jnp.int32).reshape(_N).reshape(_NW, _CHUNKS, _C)
    tabp = jnp.pad(embeddings.astype(jnp.bfloat16), ((0, 0), (0, _D)))
    out = _sc_embed(idx3, tabp)
    return out.reshape(_B, _H, _D)


# speedup vs baseline: 1.9792x; 1.9792x over previous
import functools

import jax
import jax.numpy as jnp
from jax import lax
from jax.experimental import pallas as pl
from jax.experimental.pallas import tpu as pltpu
from jax.experimental.pallas import tpu_sc as plsc

_VOCAB = 1000000
_D = 64
_B = 4096
_H = 50
_N = _B * _H
_SCALE = 8.0

_NC = 2
_NS = 16
_NW = _NC * _NS
_C = 128
_CHUNKS = _N // (_NW * _C)


@jax.jit
def _sc_embed(idx3, tabp):
    mesh = plsc.VectorSubcoreMesh(
        core_axis_name="c", subcore_axis_name="s",
        num_cores=_NC, num_subcores=_NS)

    @functools.partial(
        pl.kernel,
        out_type=jax.ShapeDtypeStruct((_N, _D), jnp.float32),
        mesh=mesh,
        scratch_types=[
            pltpu.VMEM((_CHUNKS, _C), jnp.int32),
            pltpu.VMEM((2, _C, 2 * _D), jnp.float32),
            pltpu.VMEM((2, _C, _D), jnp.float32),
            pltpu.SemaphoreType.DMA,
            pltpu.SemaphoreType.DMA,
            pltpu.SemaphoreType.DMA,
            pltpu.SemaphoreType.DMA,
        ],
        compiler_params=pltpu.CompilerParams(use_tc_tiling_on_sc=False),
    )
    def k(idx_hbm, tab_hbm, out_hbm, idx_v, rows_v, out_v,
          gsem0, gsem1, osem0, osem1):
        gsems = (gsem0, gsem1)
        osems = (osem0, osem1)
        wid = lax.axis_index("s") * _NC + lax.axis_index("c")
        pltpu.sync_copy(idx_hbm.at[wid], idx_v)

        rbase0 = pl.multiple_of(wid * (_CHUNKS * _C), _C)

        def start_gather(ck, b):
            pltpu.async_copy(tab_hbm.at[idx_v.at[ck]], rows_v.at[b], gsems[b])

        def wait_gather(ck, b):
            pltpu.make_async_copy(
                tab_hbm.at[idx_v.at[ck]], rows_v.at[b], gsems[b]).wait()

        def wb_slice(ck):
            rb = pl.multiple_of(rbase0 + ck * _C, _C)
            return out_hbm.at[pl.ds(rb, _C)]

        def extract(b):
            rows = rows_v.at[b]
            outb = out_v.at[b]

            @plsc.parallel_loop(0, _C, 1, unroll=4)
            def _(j):
                for g in range(_D // 16):
                    sl = pl.ds(16 * g, 16)
                    outb[j, sl] = rows[j, sl] * _SCALE

        start_gather(0, 0)
        start_gather(1, 1)

        @pl.loop(0, _CHUNKS // 2)
        def _(g2):
            for b in (0, 1):
                ck = 2 * g2 + b
                wait_gather(ck, b)

                @pl.when(ck >= 2)
                def _():
                    pltpu.make_async_copy(
                        out_v.at[b], wb_slice(ck - 2), osems[b]).wait()

                extract(b)
                pltpu.async_copy(out_v.at[b], wb_slice(ck), osems[b])

                @pl.when(ck + 2 < _CHUNKS)
                def _():
                    start_gather(ck + 2, b)

        pltpu.make_async_copy(
            out_v.at[0], wb_slice(_CHUNKS - 2), osems[0]).wait()
        pltpu.make_async_copy(
            out_v.at[1], wb_slice(_CHUNKS - 1), osems[1]).wait()

    return k(idx3, tabp)


def kernel(inputs, embeddings):
    idx3 = inputs.astype(jnp.int32).reshape(_N).reshape(_NW, _CHUNKS, _C)
    tabp = jnp.pad(embeddings, ((0, 0), (0, _D)))
    out = _sc_embed(idx3, tabp)
    return out.reshape(_B, _H, _D)
